# baseline (speedup 1.0000x reference)
import jax
import jax.numpy as jnp
from jax import lax
from jax.experimental import pallas as pl
from jax.experimental.pallas import tpu as pltpu
from jax.experimental.pallas import tpu_sc as plsc

B, C, T, H, W = 4, 3, 64, 224, 224
TBLK = 8
NTB = T // TBLK
HW = H * W
FS, FL = HW // 128, 128


def _bf(a):
    return a.astype(jnp.bfloat16).astype(jnp.float32)


def _conv3(x0, x1, x2, w0, w1, w2, c0):
    v = _bf(x0) * w0 + _bf(x1) * w1 + _bf(x2) * w2
    return jnp.maximum(v + c0, 0.0)


def _sum_body(x_ref, cw_ref, cb_ref, out_ref):
    b = pl.program_id(0)
    tb = pl.program_id(1)
    w0 = cw_ref[0, 0]
    w1 = cw_ref[0, 1]
    w2 = cw_ref[0, 2]
    c0 = cb_ref[0]
    v = _conv3(x_ref[0, 0], x_ref[0, 1], x_ref[0, 2], w0, w1, w2, c0)
    sums = jnp.sum(v, axis=(1, 2))
    row = jax.lax.broadcasted_iota(jnp.int32, (1, NTB, TBLK), 1)
    bcast = jnp.broadcast_to(sums[None, None, :], (1, NTB, TBLK))
    out_ref[...] = jnp.where(row == tb, bcast, out_ref[...])


def _frame_sums(x, conv_w, conv_b):
    return pl.pallas_call(
        _sum_body,
        grid=(B, NTB),
        in_specs=[
            pl.BlockSpec((1, C, TBLK, H, W), lambda b, tb: (b, 0, tb, 0, 0)),
            pl.BlockSpec(memory_space=pltpu.SMEM),
            pl.BlockSpec(memory_space=pltpu.SMEM),
        ],
        out_specs=pl.BlockSpec((1, NTB, TBLK), lambda b, tb: (b, 0, 0)),
        out_shape=jax.ShapeDtypeStruct((B, NTB, TBLK), jnp.float32),
    )(x, conv_w, conv_b)


def _score_body(y_ref, f1w_ref, f1b_ref, f2w_ref, f2b_ref, s_ref):
    y = y_ref[...] * (1.0 / HW)
    z = jax.lax.dot_general(y.astype(jnp.bfloat16), f1w_ref[...],
                            (((1,), (1,)), ((), ())),
                            preferred_element_type=jnp.float32)
    z = jnp.maximum(z + f1b_ref[...][None, :], 0.0)
    lg = jax.lax.dot_general(z.astype(jnp.bfloat16), f2w_ref[...],
                             (((1,), (1,)), ((), ())),
                             preferred_element_type=jnp.float32)
    lg = lg + f2b_ref[...][None, :]
    s_ref[...] = 1.0 / (1.0 + jnp.exp(-lg))


def _mlp_scores(y, fc1_w, fc1_b, fc2_w, fc2_b):
    return pl.pallas_call(
        _score_body,
        in_specs=[pl.BlockSpec(memory_space=pltpu.VMEM)] * 5,
        out_specs=pl.BlockSpec(memory_space=pltpu.VMEM),
        out_shape=jax.ShapeDtypeStruct((B, T), jnp.float32),
    )(y, fc1_w, fc1_b, fc2_w, fc2_b)


def _sc_topk_body(s_hbm, idx_hbm, s_v, idx_v, tf_v, ti_v):
    cid = lax.axis_index("c")
    sid = lax.axis_index("s")
    iota = lax.iota(jnp.int32, 16)
    nv = T // 16
    big_i = jnp.full((16,), 4 * T, jnp.int32)
    inf = jnp.float32(jnp.inf)

    def allmax_f(v):
        for d in (8, 4, 2, 1):
            tf_v[...] = v
            v = jnp.maximum(v, plsc.load_gather(tf_v, [iota ^ d]))
        return v

    def allmin_i(v):
        for d in (8, 4, 2, 1):
            ti_v[...] = v
            v = jnp.minimum(v, plsc.load_gather(ti_v, [iota ^ d]))
        return v

    for b in range(B):
        @pl.when(jnp.logical_and(cid == 0, sid == b))
        def _(b=b):
            pltpu.sync_copy(s_hbm.at[b], s_v)
            picks = jnp.zeros((16,), jnp.int32)
            for side in range(2):
                w = [s_v[pl.ds(j * 16, 16)] for j in range(nv)]
                for k in range(4):
                    if side == 0:
                        wred = jnp.maximum(jnp.maximum(w[0], w[1]),
                                           jnp.maximum(w[2], w[3]))
                        m = allmax_f(wred)
                    else:
                        wred = jnp.minimum(jnp.minimum(w[0], w[1]),
                                           jnp.minimum(w[2], w[3]))
                        m = -allmax_f(-wred)
                    cand = big_i
                    for j in range(nv):
                        cand = jnp.minimum(
                            cand, jnp.where(w[j] == m, iota + j * 16, big_i))
                    a = allmin_i(cand)
                    fill = -inf if side == 0 else inf
                    for j in range(nv):
                        w[j] = jnp.where(iota + j * 16 == a, fill, w[j])
                    picks = jnp.where(iota == (side * 4 + k), a, picks)
                idx_v[...] = picks
            pltpu.sync_copy(idx_v, idx_hbm.at[b])


def _sc_topk(s):
    mesh = plsc.VectorSubcoreMesh(core_axis_name="c", subcore_axis_name="s")
    fn = pl.kernel(
        _sc_topk_body,
        mesh=mesh,
        out_type=jax.ShapeDtypeStruct((B, 16), jnp.int32),
        scratch_types=[
            pltpu.VMEM((T,), jnp.float32),
            pltpu.VMEM((16,), jnp.int32),
            pltpu.VMEM((16,), jnp.float32),
            pltpu.VMEM((16,), jnp.int32),
        ],
        compiler_params=pltpu.CompilerParams(needs_layout_passes=False),
    )
    return fn(s)


def _gather_body(idx_ref, x_ref, cw_ref, cb_ref, out_ref):
    w0 = cw_ref[0, 0]
    w1 = cw_ref[0, 1]
    w2 = cw_ref[0, 2]
    c0 = cb_ref[0]
    out_ref[0, 0, 0] = _conv3(x_ref[0, 0, 0], x_ref[0, 1, 0], x_ref[0, 2, 0],
                              w0, w1, w2, c0)


def _gather(idx, x, conv_w, conv_b):
    grid_spec = pltpu.PrefetchScalarGridSpec(
        num_scalar_prefetch=1,
        grid=(B, 8),
        in_specs=[
            pl.BlockSpec((1, C, 1, H, W),
                         lambda b, j, iref: (b, 0, iref[b * 8 + j], 0, 0)),
            pl.BlockSpec(memory_space=pltpu.SMEM),
            pl.BlockSpec(memory_space=pltpu.SMEM),
        ],
        out_specs=pl.BlockSpec((1, 1, 1, H, W),
                               lambda b, j, iref: (b, 0, j, 0, 0)),
    )
    return pl.pallas_call(
        _gather_body,
        grid_spec=grid_spec,
        out_shape=jax.ShapeDtypeStruct((B, 1, 8, H, W), jnp.float32),
    )(idx, x, conv_w, conv_b)


def kernel(x, conv_w, conv_b, fc1_w, fc1_b, fc2_w, fc2_b):
    cw_r = conv_w.astype(jnp.bfloat16).astype(jnp.float32)
    part = _frame_sums(x, cw_r, conv_b)
    y = part.reshape(B, T)
    s = _mlp_scores(y, fc1_w.astype(jnp.bfloat16), fc1_b,
                    fc2_w.astype(jnp.bfloat16), fc2_b)
    idx = _sc_topk(s)[:, :8]
    return _gather(idx.reshape(-1), x, cw_r, conv_b)


# speedup vs baseline: 1.0824x; 1.0824x over previous
import jax
import jax.numpy as jnp
from jax import lax
from jax.experimental import pallas as pl
from jax.experimental.pallas import tpu as pltpu
from jax.experimental.pallas import tpu_sc as plsc

B, C, T, H, W = 4, 3, 64, 224, 224
TBLK = 16
NTB = T // TBLK
HW = H * W
FS, FL = HW // 128, 128


def _bf(a):
    return a.astype(jnp.bfloat16).astype(jnp.float32)


def _conv3(x0, x1, x2, w0, w1, w2, c0):
    v = _bf(x0) * w0 + _bf(x1) * w1 + _bf(x2) * w2
    return jnp.maximum(v + c0, 0.0)


def _sum_body(x_ref, cw_ref, cb_ref, out_ref):
    b = pl.program_id(0)
    tb = pl.program_id(1)
    w0 = cw_ref[0, 0]
    w1 = cw_ref[0, 1]
    w2 = cw_ref[0, 2]
    c0 = cb_ref[0]
    v = _conv3(x_ref[0, 0], x_ref[0, 1], x_ref[0, 2], w0, w1, w2, c0)
    sums = jnp.sum(v, axis=(1, 2))
    row = jax.lax.broadcasted_iota(jnp.int32, (1, NTB, TBLK), 1)
    bcast = jnp.broadcast_to(sums[None, None, :], (1, NTB, TBLK))
    out_ref[...] = jnp.where(row == tb, bcast, out_ref[...])


def _frame_sums(x, conv_w, conv_b):
    return pl.pallas_call(
        _sum_body,
        grid=(B, NTB),
        in_specs=[
            pl.BlockSpec((1, C, TBLK, H, W), lambda b, tb: (b, 0, tb, 0, 0)),
            pl.BlockSpec(memory_space=pltpu.SMEM),
            pl.BlockSpec(memory_space=pltpu.SMEM),
        ],
        out_specs=pl.BlockSpec((1, NTB, TBLK), lambda b, tb: (b, 0, 0)),
        out_shape=jax.ShapeDtypeStruct((B, NTB, TBLK), jnp.float32),
    )(x, conv_w, conv_b)


def _score_body(y_ref, f1w_ref, f1b_ref, f2w_ref, f2b_ref, s_ref):
    y = y_ref[...] * (1.0 / HW)
    z = jax.lax.dot_general(y.astype(jnp.bfloat16), f1w_ref[...],
                            (((1,), (1,)), ((), ())),
                            preferred_element_type=jnp.float32)
    z = jnp.maximum(z + f1b_ref[...][None, :], 0.0)
    lg = jax.lax.dot_general(z.astype(jnp.bfloat16), f2w_ref[...],
                             (((1,), (1,)), ((), ())),
                             preferred_element_type=jnp.float32)
    lg = lg + f2b_ref[...][None, :]
    s_ref[...] = 1.0 / (1.0 + jnp.exp(-lg))


def _mlp_scores(y, fc1_w, fc1_b, fc2_w, fc2_b):
    return pl.pallas_call(
        _score_body,
        in_specs=[pl.BlockSpec(memory_space=pltpu.VMEM)] * 5,
        out_specs=pl.BlockSpec(memory_space=pltpu.VMEM),
        out_shape=jax.ShapeDtypeStruct((B, T), jnp.float32),
    )(y, fc1_w, fc1_b, fc2_w, fc2_b)


def _sc_topk_body(s_hbm, idx_hbm, s_v, idx_v, tf_v, ti_v):
    cid = lax.axis_index("c")
    sid = lax.axis_index("s")
    iota = lax.iota(jnp.int32, 16)
    nv = T // 16
    big_i = jnp.full((16,), 4 * T, jnp.int32)
    inf = jnp.float32(jnp.inf)

    def allmax_f(v):
        for d in (8, 4, 2, 1):
            tf_v[...] = v
            v = jnp.maximum(v, plsc.load_gather(tf_v, [iota ^ d]))
        return v

    def allmin_i(v):
        for d in (8, 4, 2, 1):
            ti_v[...] = v
            v = jnp.minimum(v, plsc.load_gather(ti_v, [iota ^ d]))
        return v

    for b in range(B):
        @pl.when(jnp.logical_and(cid == 0, sid == b))
        def _(b=b):
            pltpu.sync_copy(s_hbm.at[b], s_v)
            picks = jnp.zeros((16,), jnp.int32)
            for side in range(2):
                w = [s_v[pl.ds(j * 16, 16)] for j in range(nv)]
                for k in range(4):
                    if side == 0:
                        wred = jnp.maximum(jnp.maximum(w[0], w[1]),
                                           jnp.maximum(w[2], w[3]))
                        m = allmax_f(wred)
                    else:
                        wred = jnp.minimum(jnp.minimum(w[0], w[1]),
                                           jnp.minimum(w[2], w[3]))
                        m = -allmax_f(-wred)
                    cand = big_i
                    for j in range(nv):
                        cand = jnp.minimum(
                            cand, jnp.where(w[j] == m, iota + j * 16, big_i))
                    a = allmin_i(cand)
                    fill = -inf if side == 0 else inf
                    for j in range(nv):
                        w[j] = jnp.where(iota + j * 16 == a, fill, w[j])
                    picks = jnp.where(iota == (side * 4 + k), a, picks)
                idx_v[...] = picks
            pltpu.sync_copy(idx_v, idx_hbm.at[b])


def _sc_topk(s):
    mesh = plsc.VectorSubcoreMesh(core_axis_name="c", subcore_axis_name="s")
    fn = pl.kernel(
        _sc_topk_body,
        mesh=mesh,
        out_type=jax.ShapeDtypeStruct((B, 16), jnp.int32),
        scratch_types=[
            pltpu.VMEM((T,), jnp.float32),
            pltpu.VMEM((16,), jnp.int32),
            pltpu.VMEM((16,), jnp.float32),
            pltpu.VMEM((16,), jnp.int32),
        ],
        compiler_params=pltpu.CompilerParams(needs_layout_passes=False),
    )
    return fn(s)


def _gather_body(idx_ref, x_ref, cw_ref, cb_ref, out_ref):
    w0 = cw_ref[0, 0]
    w1 = cw_ref[0, 1]
    w2 = cw_ref[0, 2]
    c0 = cb_ref[0]
    out_ref[0, 0, 0] = _conv3(x_ref[0, 0, 0], x_ref[0, 1, 0], x_ref[0, 2, 0],
                              w0, w1, w2, c0)


def _gather(idx, x, conv_w, conv_b):
    grid_spec = pltpu.PrefetchScalarGridSpec(
        num_scalar_prefetch=1,
        grid=(B, 8),
        in_specs=[
            pl.BlockSpec((1, C, 1, H, W),
                         lambda b, j, iref: (b, 0, iref[b * 8 + j], 0, 0)),
            pl.BlockSpec(memory_space=pltpu.SMEM),
            pl.BlockSpec(memory_space=pltpu.SMEM),
        ],
        out_specs=pl.BlockSpec((1, 1, 1, H, W),
                               lambda b, j, iref: (b, 0, j, 0, 0)),
    )
    return pl.pallas_call(
        _gather_body,
        grid_spec=grid_spec,
        out_shape=jax.ShapeDtypeStruct((B, 1, 8, H, W), jnp.float32),
    )(idx, x, conv_w, conv_b)


def kernel(x, conv_w, conv_b, fc1_w, fc1_b, fc2_w, fc2_b):
    cw_r = conv_w.astype(jnp.bfloat16).astype(jnp.float32)
    part = _frame_sums(x, cw_r, conv_b)
    y = part.reshape(B, T)
    s = _mlp_scores(y, fc1_w.astype(jnp.bfloat16), fc1_b,
                    fc2_w.astype(jnp.bfloat16), fc2_b)
    idx = _sc_topk(s)[:, :8]
    return _gather(idx.reshape(-1), x, cw_r, conv_b)
